# Initial kernel scaffold; baseline (speedup 1.0000x reference)
import jax
import jax.numpy as jnp
from jax.experimental import pallas as pl


def kernel(x, pos, edge_index, batch, embedding, mlp_W1, mlp_b1, mlp_W2, mlp_b2, lin1_W, lin2_W, lin2_b, lin_W, lin_b, out_W1, out_b1, out_W2, out_b2):
    raise NotImplementedError("write your pallas kernel here")


# speedup vs baseline: 1.4022x; 1.4022x over previous
import functools
import math

import jax
import jax.numpy as jnp
from jax import lax
from jax.experimental import pallas as pl
from jax.experimental.pallas import tpu as pltpu
from jax.experimental.pallas import tpu_sc as plsc

_N = 10000
_E = 160000
_H = 64
_F = 64
_G = 50
_NG = 16
_L = 3
_CUT = 10.0

_CH = 128
_NCHUNK = _E // _CH
_NTILE = 32
_RPT = 624
_RTAIL = _N - 16 * _RPT


def _ssp(v):
    return jnp.logaddexp(v, 0.0) - math.log(2.0)


def _sc_edge_d2(pos_flat, src, dst):
    mesh = plsc.VectorSubcoreMesh(core_axis_name="c", subcore_axis_name="s")

    @functools.partial(
        pl.kernel,
        mesh=mesh,
        out_type=jax.ShapeDtypeStruct((_E,), jnp.float32),
        scratch_types=[
            pltpu.VMEM((_N * 3,), jnp.float32),
            pltpu.VMEM((_CH,), jnp.int32),
            pltpu.VMEM((_CH,), jnp.int32),
            pltpu.VMEM((_CH,), jnp.float32),
        ],
        compiler_params=pltpu.CompilerParams(needs_layout_passes=False),
    )
    def k(pos_hbm, src_hbm, dst_hbm, out_hbm, pos_v, src_v, dst_v, d2_v):
        c = lax.axis_index("c")
        s = lax.axis_index("s")
        wid = s * 2 + c
        pltpu.sync_copy(pos_hbm, pos_v)
        nc = _NCHUNK // _NTILE + jnp.where(wid < _NCHUNK % _NTILE, 1, 0)

        @pl.loop(0, nc)
        def _(kk):
            base = (wid + kk * _NTILE) * _CH
            pltpu.sync_copy(src_hbm.at[pl.ds(base, _CH)], src_v)
            pltpu.sync_copy(dst_hbm.at[pl.ds(base, _CH)], dst_v)

            @pl.loop(0, _CH // 16)
            def _(g):
                sl = pl.ds(g * 16, 16)
                si = src_v[sl] * 3
                di = dst_v[sl] * 3
                acc = jnp.zeros((16,), jnp.float32)
                for kcoord in range(3):
                    ps = plsc.load_gather(pos_v, [si + kcoord])
                    pd = plsc.load_gather(pos_v, [di + kcoord])
                    df = pd - ps
                    acc = acc + df * df
                d2_v[sl] = acc

            pltpu.sync_copy(d2_v, out_hbm.at[pl.ds(base, _CH)])

    return k(pos_flat, src, dst)


def _sc_message(hlin, w, src, dst, zeros):
    mesh = plsc.VectorSubcoreMesh(core_axis_name="c", subcore_axis_name="s")

    @functools.partial(
        pl.kernel,
        mesh=mesh,
        out_type=jax.ShapeDtypeStruct((2, _N, _F), jnp.float32),
        scratch_types=[
            pltpu.VMEM_SHARED((_N, _F), jnp.float32),
            pltpu.VMEM_SHARED((_N, _F), jnp.float32),
            pltpu.VMEM((_CH,), jnp.int32),
            pltpu.VMEM((_CH,), jnp.int32),
            pltpu.VMEM((_CH, _F), jnp.float32),
            pltpu.VMEM((_CH, _F), jnp.float32),
        ],
        compiler_params=pltpu.CompilerParams(
            needs_layout_passes=False, use_tc_tiling_on_sc=False),
    )
    def k(hlin_hbm, w_hbm, src_hbm, dst_hbm, zero_hbm, out_hbm,
          hlin_s, agg_s, src_v, dst_v, w_v, rows_v):
        c = lax.axis_index("c")
        s = lax.axis_index("s")
        wid = s * 2 + c
        rsl = pl.ds(s * _RPT, _RPT)
        pltpu.sync_copy(hlin_hbm.at[rsl], hlin_s.at[rsl])
        pltpu.sync_copy(zero_hbm.at[rsl], agg_s.at[rsl])

        @pl.when(s == 0)
        def _():
            tsl = pl.ds(16 * _RPT, _RTAIL)
            pltpu.sync_copy(hlin_hbm.at[tsl], hlin_s.at[tsl])
            pltpu.sync_copy(zero_hbm.at[tsl], agg_s.at[tsl])

        plsc.subcore_barrier()

        nc = _NCHUNK // _NTILE + jnp.where(wid < _NCHUNK % _NTILE, 1, 0)

        @pl.loop(0, nc)
        def _(kk):
            base = (wid + kk * _NTILE) * _CH
            pltpu.sync_copy(src_hbm.at[pl.ds(base, _CH)], src_v)
            pltpu.sync_copy(dst_hbm.at[pl.ds(base, _CH)], dst_v)
            pltpu.sync_copy(w_hbm.at[pl.ds(base, _CH)], w_v)
            pltpu.sync_copy(hlin_s.at[src_v], rows_v)

            @pl.loop(0, _CH)
            def _(r):
                for j in range(_F // 16):
                    sl = pl.ds(j * 16, 16)
                    rows_v[r, sl] = rows_v[r, sl] * w_v[r, sl]

            pltpu.sync_copy(rows_v, agg_s.at[dst_v], add=True)

        plsc.subcore_barrier()
        pltpu.sync_copy(agg_s.at[rsl], out_hbm.at[c].at[rsl])

        @pl.when(s == 0)
        def _():
            tsl = pl.ds(16 * _RPT, _RTAIL)
            pltpu.sync_copy(agg_s.at[tsl], out_hbm.at[c].at[tsl])

    return k(hlin, w, src, dst, zeros)


_BN = 1000
_BE = 1000


def _embed_call(x2, embedding, lin1_0):
    def body(x_ref, emb_ref, l1_ref, h_ref, hlin_ref):
        xb = x_ref[...]
        oh = (xb == lax.broadcasted_iota(jnp.int32, (_BN, 100), 1))
        oh = oh.astype(jnp.float32)
        h = jnp.dot(oh, emb_ref[...], preferred_element_type=jnp.float32, precision=lax.Precision.HIGHEST)
        h_ref[...] = h
        hlin_ref[...] = jnp.dot(h, l1_ref[...],
                                preferred_element_type=jnp.float32, precision=lax.Precision.HIGHEST)

    return pl.pallas_call(
        body,
        grid=(_N // _BN,),
        in_specs=[
            pl.BlockSpec((_BN, 1), lambda i: (i, 0)),
            pl.BlockSpec((100, _H), lambda i: (0, 0)),
            pl.BlockSpec((_H, _F), lambda i: (0, 0)),
        ],
        out_specs=[
            pl.BlockSpec((_BN, _H), lambda i: (i, 0)),
            pl.BlockSpec((_BN, _F), lambda i: (i, 0)),
        ],
        out_shape=[
            jax.ShapeDtypeStruct((_N, _H), jnp.float32),
            jax.ShapeDtypeStruct((_N, _F), jnp.float32),
        ],
    )(x2, embedding, lin1_0)


def _filter_call(d2, mlp_W1, mlp_b1, mlp_W2, mlp_b2):
    dg = _CUT / (_G - 1)
    coeff = -0.5 / dg**2

    def body(d2_ref, w1_ref, b1_ref, w2_ref, b2_ref, out_ref):
        d = jnp.sqrt(d2_ref[...] + 1e-12)
        offs = lax.broadcasted_iota(jnp.int32, (1, _G), 1).astype(
            jnp.float32) * dg
        rbf = jnp.exp(coeff * (d - offs) ** 2)
        cc = 0.5 * (jnp.cos(d * (math.pi / _CUT)) + 1.0)
        cc = cc * (d < _CUT).astype(jnp.float32)
        for i in range(_L):
            t = jnp.dot(rbf, w1_ref[i], preferred_element_type=jnp.float32, precision=lax.Precision.HIGHEST)
            t = _ssp(t + b1_ref[i])
            w = jnp.dot(t, w2_ref[i], preferred_element_type=jnp.float32, precision=lax.Precision.HIGHEST)
            out_ref[i] = (w + b2_ref[i]) * cc

    return pl.pallas_call(
        body,
        grid=(_E // _BE,),
        in_specs=[
            pl.BlockSpec((_BE, 1), lambda i: (i, 0)),
            pl.BlockSpec((_L, _G, _F), lambda i: (0, 0, 0)),
            pl.BlockSpec((_L, 1, _F), lambda i: (0, 0, 0)),
            pl.BlockSpec((_L, _F, _F), lambda i: (0, 0, 0)),
            pl.BlockSpec((_L, 1, _F), lambda i: (0, 0, 0)),
        ],
        out_specs=pl.BlockSpec((_L, _BE, _F), lambda i: (0, i, 0)),
        out_shape=jax.ShapeDtypeStruct((_L, _E, _F), jnp.float32),
    )(d2, mlp_W1, mlp_b1, mlp_W2, mlp_b2)


def _update_call(agg2, h, lin2_Wi, lin2_bi, lin_Wi, lin_bi, lin1_next):
    def body(a_ref, h_ref, w2_ref, b2_ref, w_ref, b_ref, l1_ref,
             hn_ref, hlin_ref):
        agg = a_ref[0] + a_ref[1]
        v = _ssp(jnp.dot(agg, w2_ref[...],
                         preferred_element_type=jnp.float32, precision=lax.Precision.HIGHEST) + b2_ref[...])
        v = jnp.dot(v, w_ref[...], preferred_element_type=jnp.float32, precision=lax.Precision.HIGHEST)
        v = v + b_ref[...]
        hn = h_ref[...] + v
        hn_ref[...] = hn
        hlin_ref[...] = jnp.dot(hn, l1_ref[...],
                                preferred_element_type=jnp.float32, precision=lax.Precision.HIGHEST)

    return pl.pallas_call(
        body,
        grid=(_N // _BN,),
        in_specs=[
            pl.BlockSpec((2, _BN, _F), lambda i: (0, i, 0)),
            pl.BlockSpec((_BN, _H), lambda i: (i, 0)),
            pl.BlockSpec((_F, _H), lambda i: (0, 0)),
            pl.BlockSpec((1, _H), lambda i: (0, 0)),
            pl.BlockSpec((_H, _H), lambda i: (0, 0)),
            pl.BlockSpec((1, _H), lambda i: (0, 0)),
            pl.BlockSpec((_H, _F), lambda i: (0, 0)),
        ],
        out_specs=[
            pl.BlockSpec((_BN, _H), lambda i: (i, 0)),
            pl.BlockSpec((_BN, _F), lambda i: (i, 0)),
        ],
        out_shape=[
            jax.ShapeDtypeStruct((_N, _H), jnp.float32),
            jax.ShapeDtypeStruct((_N, _F), jnp.float32),
        ],
    )(agg2, h, lin2_Wi, lin2_bi, lin_Wi, lin_bi, lin1_next)


def _readout_call(batch2, h, ow1, ob1, ow2, ob2):
    nblk = _N // _BN

    def body(b_ref, h_ref, w1_ref, b1_ref, w2_ref, b2_ref, out_ref, g_acc):
        i = pl.program_id(0)

        @pl.when(i == 0)
        def _():
            g_acc[...] = jnp.zeros((_NG, _H), jnp.float32)

        oh = (b_ref[...] == lax.broadcasted_iota(jnp.int32, (_BN, _NG), 1))
        oh = oh.astype(jnp.float32)
        g_acc[...] += lax.dot_general(
            oh, h_ref[...], (((0,), (0,)), ((), ())),
            preferred_element_type=jnp.float32, precision=lax.Precision.HIGHEST)

        @pl.when(i == nblk - 1)
        def _():
            g = g_acc[...]
            z = jnp.maximum(
                jnp.dot(g, w1_ref[...], preferred_element_type=jnp.float32, precision=lax.Precision.HIGHEST)
                + b1_ref[...], 0.0)
            out_ref[...] = (
                jnp.dot(z, w2_ref[...], preferred_element_type=jnp.float32, precision=lax.Precision.HIGHEST)
                + b2_ref[...])

    return pl.pallas_call(
        body,
        grid=(nblk,),
        in_specs=[
            pl.BlockSpec((_BN, 1), lambda i: (i, 0)),
            pl.BlockSpec((_BN, _H), lambda i: (i, 0)),
            pl.BlockSpec((_H, _H // 2), lambda i: (0, 0)),
            pl.BlockSpec((1, _H // 2), lambda i: (0, 0)),
            pl.BlockSpec((_H // 2, 1), lambda i: (0, 0)),
            pl.BlockSpec((1, 1), lambda i: (0, 0)),
        ],
        out_specs=pl.BlockSpec((_NG, 1), lambda i: (0, 0)),
        out_shape=jax.ShapeDtypeStruct((_NG, 1), jnp.float32),
        scratch_shapes=[pltpu.VMEM((_NG, _H), jnp.float32)],
    )(batch2, h, ow1, ob1, ow2, ob2)


def kernel(x, pos, edge_index, batch, embedding, mlp_W1, mlp_b1, mlp_W2,
           mlp_b2, lin1_W, lin2_W, lin2_b, lin_W, lin_b, out_W1, out_b1,
           out_W2, out_b2):
    dst = edge_index[0].astype(jnp.int32)
    src = edge_index[1].astype(jnp.int32)
    x2 = x.astype(jnp.int32).reshape(_N, 1)
    batch2 = batch.astype(jnp.int32).reshape(_N, 1)
    pos_flat = pos.reshape(_N * 3)
    zeros = jnp.zeros((_N, _F), jnp.float32)

    d2 = _sc_edge_d2(pos_flat, src, dst)
    w_all = _filter_call(d2.reshape(_E, 1), mlp_W1,
                         mlp_b1.reshape(_L, 1, _F), mlp_W2,
                         mlp_b2.reshape(_L, 1, _F))
    h, hlin = _embed_call(x2, embedding, lin1_W[0])
    for i in range(_L):
        agg2 = _sc_message(hlin, w_all[i], src, dst, zeros)
        h, hlin = _update_call(agg2, h, lin2_W[i],
                               lin2_b[i].reshape(1, _H), lin_W[i],
                               lin_b[i].reshape(1, _H),
                               lin1_W[(i + 1) % _L])
    out = _readout_call(batch2, h, out_W1, out_b1.reshape(1, _H // 2),
                        out_W2, out_b2.reshape(1, 1))
    return out.reshape(_NG)


# speedup vs baseline: 1.7551x; 1.2517x over previous
import functools
import math

import jax
import jax.numpy as jnp
from jax import lax
from jax.experimental import pallas as pl
from jax.experimental.pallas import tpu as pltpu
from jax.experimental.pallas import tpu_sc as plsc

_N = 10000
_E = 160000
_H = 64
_F = 64
_G = 50
_NG = 16
_L = 3
_CUT = 10.0

_CH = 128
_NCHUNK = _E // _CH
_NTILE = 32
_RPT = 624
_RTAIL = _N - 16 * _RPT


def _ssp(v):
    return (jnp.maximum(v, 0.0) + jnp.log1p(jnp.exp(-jnp.abs(v)))
            - math.log(2.0))


def _sc_edge_d2(pos_flat, src, dst):
    mesh = plsc.VectorSubcoreMesh(core_axis_name="c", subcore_axis_name="s")

    @functools.partial(
        pl.kernel,
        mesh=mesh,
        out_type=jax.ShapeDtypeStruct((_E,), jnp.float32),
        scratch_types=[
            pltpu.VMEM((_N * 3,), jnp.float32),
            pltpu.VMEM((_CH,), jnp.int32),
            pltpu.VMEM((_CH,), jnp.int32),
            pltpu.VMEM((_CH,), jnp.float32),
        ],
        compiler_params=pltpu.CompilerParams(needs_layout_passes=False),
    )
    def k(pos_hbm, src_hbm, dst_hbm, out_hbm, pos_v, src_v, dst_v, d2_v):
        c = lax.axis_index("c")
        s = lax.axis_index("s")
        wid = s * 2 + c
        pltpu.sync_copy(pos_hbm, pos_v)
        nc = _NCHUNK // _NTILE + jnp.where(wid < _NCHUNK % _NTILE, 1, 0)

        @pl.loop(0, nc)
        def _(kk):
            base = (wid + kk * _NTILE) * _CH
            pltpu.sync_copy(src_hbm.at[pl.ds(base, _CH)], src_v)
            pltpu.sync_copy(dst_hbm.at[pl.ds(base, _CH)], dst_v)

            @pl.loop(0, _CH // 16)
            def _(g):
                sl = pl.ds(g * 16, 16)
                si = src_v[sl] * 3
                di = dst_v[sl] * 3
                acc = jnp.zeros((16,), jnp.float32)
                for kcoord in range(3):
                    ps = plsc.load_gather(pos_v, [si + kcoord])
                    pd = plsc.load_gather(pos_v, [di + kcoord])
                    df = pd - ps
                    acc = acc + df * df
                d2_v[sl] = acc

            pltpu.sync_copy(d2_v, out_hbm.at[pl.ds(base, _CH)])

    return k(pos_flat, src, dst)


def _sc_message(hlin, w, src, dst, zeros):
    mesh = plsc.VectorSubcoreMesh(core_axis_name="c", subcore_axis_name="s")

    @functools.partial(
        pl.kernel,
        mesh=mesh,
        out_type=jax.ShapeDtypeStruct((2, _N, _F), jnp.float32),
        scratch_types=[
            pltpu.VMEM_SHARED((_N, _F), jnp.float32),
            pltpu.VMEM_SHARED((_N, _F), jnp.float32),
            pltpu.VMEM((_CH,), jnp.int32),
            pltpu.VMEM((_CH,), jnp.int32),
            pltpu.VMEM((_CH, _F), jnp.float32),
            pltpu.VMEM((_CH, _F), jnp.float32),
        ],
        compiler_params=pltpu.CompilerParams(
            needs_layout_passes=False, use_tc_tiling_on_sc=False),
    )
    def k(hlin_hbm, w_hbm, src_hbm, dst_hbm, zero_hbm, out_hbm,
          hlin_s, agg_s, src_v, dst_v, w_v, rows_v):
        c = lax.axis_index("c")
        s = lax.axis_index("s")
        wid = s * 2 + c
        rsl = pl.ds(s * _RPT, _RPT)
        pltpu.sync_copy(hlin_hbm.at[rsl], hlin_s.at[rsl])
        pltpu.sync_copy(zero_hbm.at[rsl], agg_s.at[rsl])

        @pl.when(s == 0)
        def _():
            tsl = pl.ds(16 * _RPT, _RTAIL)
            pltpu.sync_copy(hlin_hbm.at[tsl], hlin_s.at[tsl])
            pltpu.sync_copy(zero_hbm.at[tsl], agg_s.at[tsl])

        plsc.subcore_barrier()

        nc = _NCHUNK // _NTILE + jnp.where(wid < _NCHUNK % _NTILE, 1, 0)

        @pl.loop(0, nc)
        def _(kk):
            base = (wid + kk * _NTILE) * _CH
            pltpu.sync_copy(src_hbm.at[pl.ds(base, _CH)], src_v)
            pltpu.sync_copy(dst_hbm.at[pl.ds(base, _CH)], dst_v)
            pltpu.sync_copy(w_hbm.at[pl.ds(base, _CH)], w_v)
            pltpu.sync_copy(hlin_s.at[src_v], rows_v)

            @pl.loop(0, _CH)
            def _(r):
                for j in range(_F // 16):
                    sl = pl.ds(j * 16, 16)
                    rows_v[r, sl] = rows_v[r, sl] * w_v[r, sl]

            pltpu.sync_copy(rows_v, agg_s.at[dst_v], add=True)

        plsc.subcore_barrier()
        pltpu.sync_copy(agg_s.at[rsl], out_hbm.at[c].at[rsl])

        @pl.when(s == 0)
        def _():
            tsl = pl.ds(16 * _RPT, _RTAIL)
            pltpu.sync_copy(agg_s.at[tsl], out_hbm.at[c].at[tsl])

    return k(hlin, w, src, dst, zeros)


_BN = 1000
_BE = 1000


def _embed_call(x2, embedding, lin1_0):
    def body(x_ref, emb_ref, l1_ref, h_ref, hlin_ref):
        xb = x_ref[...]
        oh = (xb == lax.broadcasted_iota(jnp.int32, (_BN, 100), 1))
        oh = oh.astype(jnp.float32)
        h = jnp.dot(oh, emb_ref[...], preferred_element_type=jnp.float32, precision=lax.Precision.HIGHEST)
        h_ref[...] = h
        hlin_ref[...] = jnp.dot(h, l1_ref[...],
                                preferred_element_type=jnp.float32, precision=lax.Precision.HIGHEST)

    return pl.pallas_call(
        body,
        grid=(_N // _BN,),
        in_specs=[
            pl.BlockSpec((_BN, 1), lambda i: (i, 0)),
            pl.BlockSpec((100, _H), lambda i: (0, 0)),
            pl.BlockSpec((_H, _F), lambda i: (0, 0)),
        ],
        out_specs=[
            pl.BlockSpec((_BN, _H), lambda i: (i, 0)),
            pl.BlockSpec((_BN, _F), lambda i: (i, 0)),
        ],
        out_shape=[
            jax.ShapeDtypeStruct((_N, _H), jnp.float32),
            jax.ShapeDtypeStruct((_N, _F), jnp.float32),
        ],
    )(x2, embedding, lin1_0)


def _filter_call(d2, w1c, b1c, w2bd, b2c):
    dg = _CUT / (_G - 1)
    coeff = -0.5 / dg**2
    lf = _L * _F

    def body(d2_ref, w1_ref, b1_ref, w2_ref, b2_ref, out_ref):
        d = jnp.sqrt(d2_ref[...] + 1e-12)
        offs = lax.broadcasted_iota(jnp.int32, (1, _G), 1).astype(
            jnp.float32) * dg
        rbf = jnp.exp(coeff * (d - offs) ** 2)
        cc = 0.5 * (jnp.cos(d * (math.pi / _CUT)) + 1.0)
        cc = cc * (d < _CUT).astype(jnp.float32)
        t = _ssp(jnp.dot(rbf, w1_ref[...],
                         preferred_element_type=jnp.float32,
                         precision=lax.Precision.HIGHEST) + b1_ref[...])
        w = jnp.dot(t, w2_ref[...], preferred_element_type=jnp.float32,
                    precision=lax.Precision.HIGHEST) + b2_ref[...]
        w = w * cc
        for i in range(_L):
            out_ref[i] = w[:, i * _F:(i + 1) * _F]

    return pl.pallas_call(
        body,
        grid=(_E // _BE,),
        in_specs=[
            pl.BlockSpec((_BE, 1), lambda i: (i, 0)),
            pl.BlockSpec((_G, lf), lambda i: (0, 0)),
            pl.BlockSpec((1, lf), lambda i: (0, 0)),
            pl.BlockSpec((lf, lf), lambda i: (0, 0)),
            pl.BlockSpec((1, lf), lambda i: (0, 0)),
        ],
        out_specs=pl.BlockSpec((_L, _BE, _F), lambda i: (0, i, 0)),
        out_shape=jax.ShapeDtypeStruct((_L, _E, _F), jnp.float32),
    )(d2, w1c, b1c, w2bd, b2c)


def _update_call(agg2, h, lin2_Wi, lin2_bi, lin_Wi, lin_bi, lin1_next):
    def body(a_ref, h_ref, w2_ref, b2_ref, w_ref, b_ref, l1_ref,
             hn_ref, hlin_ref):
        agg = a_ref[0] + a_ref[1]
        v = _ssp(jnp.dot(agg, w2_ref[...],
                         preferred_element_type=jnp.float32, precision=lax.Precision.HIGHEST) + b2_ref[...])
        v = jnp.dot(v, w_ref[...], preferred_element_type=jnp.float32, precision=lax.Precision.HIGHEST)
        v = v + b_ref[...]
        hn = h_ref[...] + v
        hn_ref[...] = hn
        hlin_ref[...] = jnp.dot(hn, l1_ref[...],
                                preferred_element_type=jnp.float32, precision=lax.Precision.HIGHEST)

    return pl.pallas_call(
        body,
        grid=(_N // _BN,),
        in_specs=[
            pl.BlockSpec((2, _BN, _F), lambda i: (0, i, 0)),
            pl.BlockSpec((_BN, _H), lambda i: (i, 0)),
            pl.BlockSpec((_F, _H), lambda i: (0, 0)),
            pl.BlockSpec((1, _H), lambda i: (0, 0)),
            pl.BlockSpec((_H, _H), lambda i: (0, 0)),
            pl.BlockSpec((1, _H), lambda i: (0, 0)),
            pl.BlockSpec((_H, _F), lambda i: (0, 0)),
        ],
        out_specs=[
            pl.BlockSpec((_BN, _H), lambda i: (i, 0)),
            pl.BlockSpec((_BN, _F), lambda i: (i, 0)),
        ],
        out_shape=[
            jax.ShapeDtypeStruct((_N, _H), jnp.float32),
            jax.ShapeDtypeStruct((_N, _F), jnp.float32),
        ],
    )(agg2, h, lin2_Wi, lin2_bi, lin_Wi, lin_bi, lin1_next)


def _readout_call(batch2, h, ow1, ob1, ow2, ob2):
    nblk = _N // _BN

    def body(b_ref, h_ref, w1_ref, b1_ref, w2_ref, b2_ref, out_ref, g_acc):
        i = pl.program_id(0)

        @pl.when(i == 0)
        def _():
            g_acc[...] = jnp.zeros((_NG, _H), jnp.float32)

        oh = (b_ref[...] == lax.broadcasted_iota(jnp.int32, (_BN, _NG), 1))
        oh = oh.astype(jnp.float32)
        g_acc[...] += lax.dot_general(
            oh, h_ref[...], (((0,), (0,)), ((), ())),
            preferred_element_type=jnp.float32, precision=lax.Precision.HIGHEST)

        @pl.when(i == nblk - 1)
        def _():
            g = g_acc[...]
            z = jnp.maximum(
                jnp.dot(g, w1_ref[...], preferred_element_type=jnp.float32, precision=lax.Precision.HIGHEST)
                + b1_ref[...], 0.0)
            out_ref[...] = (
                jnp.dot(z, w2_ref[...], preferred_element_type=jnp.float32, precision=lax.Precision.HIGHEST)
                + b2_ref[...])

    return pl.pallas_call(
        body,
        grid=(nblk,),
        in_specs=[
            pl.BlockSpec((_BN, 1), lambda i: (i, 0)),
            pl.BlockSpec((_BN, _H), lambda i: (i, 0)),
            pl.BlockSpec((_H, _H // 2), lambda i: (0, 0)),
            pl.BlockSpec((1, _H // 2), lambda i: (0, 0)),
            pl.BlockSpec((_H // 2, 1), lambda i: (0, 0)),
            pl.BlockSpec((1, 1), lambda i: (0, 0)),
        ],
        out_specs=pl.BlockSpec((_NG, 1), lambda i: (0, 0)),
        out_shape=jax.ShapeDtypeStruct((_NG, 1), jnp.float32),
        scratch_shapes=[pltpu.VMEM((_NG, _H), jnp.float32)],
    )(batch2, h, ow1, ob1, ow2, ob2)


def kernel(x, pos, edge_index, batch, embedding, mlp_W1, mlp_b1, mlp_W2,
           mlp_b2, lin1_W, lin2_W, lin2_b, lin_W, lin_b, out_W1, out_b1,
           out_W2, out_b2):
    dst = edge_index[0].astype(jnp.int32)
    src = edge_index[1].astype(jnp.int32)
    x2 = x.astype(jnp.int32).reshape(_N, 1)
    batch2 = batch.astype(jnp.int32).reshape(_N, 1)
    pos_flat = pos.reshape(_N * 3)
    zeros = jnp.zeros((_N, _F), jnp.float32)

    d2 = _sc_edge_d2(pos_flat, src, dst)
    lf = _L * _F
    w1c = jnp.transpose(mlp_W1, (1, 0, 2)).reshape(_G, lf)
    b1c = mlp_b1.reshape(1, lf)
    w2bd = jnp.zeros((lf, lf), jnp.float32)
    for i in range(_L):
        w2bd = w2bd.at[i * _F:(i + 1) * _F, i * _F:(i + 1) * _F].set(
            mlp_W2[i])
    b2c = mlp_b2.reshape(1, lf)
    w_all = _filter_call(d2.reshape(_E, 1), w1c, b1c, w2bd, b2c)
    h, hlin = _embed_call(x2, embedding, lin1_W[0])
    for i in range(_L):
        agg2 = _sc_message(hlin, w_all[i], src, dst, zeros)
        h, hlin = _update_call(agg2, h, lin2_W[i],
                               lin2_b[i].reshape(1, _H), lin_W[i],
                               lin_b[i].reshape(1, _H),
                               lin1_W[(i + 1) % _L])
    out = _readout_call(batch2, h, out_W1, out_b1.reshape(1, _H // 2),
                        out_W2, out_b2.reshape(1, 1))
    return out.reshape(_NG)
